# Initial kernel scaffold; baseline (speedup 1.0000x reference)
import jax
import jax.numpy as jnp
from jax.experimental import pallas as pl


def kernel(ligand_x, ligand_pos, protein_x, protein_pos, x_eps, pos_eps_raw, W1, b1, W2x, b2x, W2pos, b2pos, ligand_batch, protein_batch, t):
    raise NotImplementedError("write your pallas kernel here")


# speedup vs baseline: 3.9890x; 3.9890x over previous
import functools

import jax
import jax.numpy as jnp
import numpy as np
from jax.experimental import pallas as pl
from jax.experimental.pallas import tpu as pltpu

T = 200
NUM_GRAPHS = 512
D_FEAT = 128
HIDDEN = 64

_INTERPRET = False

def _sched_table():
    tt = np.arange(T + 1, dtype=np.float64)
    alpha_bar = (1.0 - (tt / T) ** 2.0) ** 2
    alpha_bar = np.clip(alpha_bar, 1e-4, 1.0)
    alpha = np.clip(alpha_bar[1:] / alpha_bar[:-1], 1e-3, 1.0)
    alpha_bar = np.cumprod(alpha)
    sab = np.sqrt(alpha_bar)
    somab = np.sqrt(1.0 - alpha_bar)
    out = np.zeros((256, 2), np.float32)
    out[:T, 0] = sab
    out[:T, 1] = somab
    return jnp.asarray(out)

_SCHED = _sched_table()

_HI = jax.lax.Precision.HIGHEST


def _dot(a, b):
    return jnp.dot(a, b, precision=_HI, preferred_element_type=jnp.float32)


def _segsum_body(segs_ref, p8_ref, out_ref):
    i = pl.program_id(0)
    seg_row = segs_ref[...].reshape(1, -1)
    iota_g = jax.lax.broadcasted_iota(jnp.int32, (NUM_GRAPHS, 1), 0).astype(jnp.float32)
    onehot_t = (iota_g == seg_row).astype(jnp.float32)
    part = _dot(onehot_t, p8_ref[...])

    @pl.when(i == 0)
    def _init():
        out_ref[...] = part[None]

    @pl.when(i != 0)
    def _acc():
        out_ref[...] = out_ref[...] + part[None]


def _segment_sums_tc(p8, segs3, block):
    n = p8.shape[0]
    nb = n // block
    return pl.pallas_call(
        _segsum_body,
        grid=(nb,),
        in_specs=[
            pl.BlockSpec((1, 1, block), lambda i: (i, 0, 0)),
            pl.BlockSpec((block, 8), lambda i: (i, 0)),
        ],
        out_specs=pl.BlockSpec((1, NUM_GRAPHS, 8), lambda i: (0, 0, 0)),
        out_shape=jax.ShapeDtypeStruct((1, NUM_GRAPHS, 8), jnp.float32),
        compiler_params=pltpu.CompilerParams(
            dimension_semantics=("arbitrary",)),
        interpret=_INTERPRET,
    )(segs3, p8)


def _main_body(parts_ref, t_ref, sched_ref, w1a_ref, w1b_ref, b1_ref,
               w2x_ref, b2x_ref, w2p_ref, b2p_ref,
               lx_ref, xe_ref, p8_ref,
               ox_ref, op_ref, tbl_ref):
    i = pl.program_id(0)

    @pl.when(i == 0)
    def _finalize_table():
        sums = jnp.sum(parts_ref[...], axis=0)
        cnt = jnp.maximum(sums[:, 6:7], 1.0)
        means = sums[:, 0:6] / cnt
        t_f = t_ref[...].astype(jnp.float32)
        iota_t = jax.lax.broadcasted_iota(jnp.int32, (1, 256), 1).astype(jnp.float32)
        onehot_t = (t_f == iota_t).astype(jnp.float32)
        sch = _dot(onehot_t, sched_ref[...])
        tfeat = t_f * (1.0 / T)
        pad = jnp.zeros((NUM_GRAPHS, 7), jnp.float32)
        tbl_ref[...] = jnp.concatenate([means, sch, tfeat, pad], axis=1)

    p8 = p8_ref[...]
    seg_col = p8[:, 7:8]
    iota_g = jax.lax.broadcasted_iota(jnp.int32, (1, NUM_GRAPHS), 1).astype(jnp.float32)
    onehot = (seg_col == iota_g).astype(jnp.float32)
    vals = _dot(onehot, tbl_ref[...])

    mean_pos = vals[:, 0:3]
    mean_eps = vals[:, 3:6]
    sab = vals[:, 6:7]
    somab = vals[:, 7:8]
    tfeat = vals[:, 8:9]

    pos = p8[:, 0:3]
    eps = p8[:, 3:6]
    pos_eps = eps - mean_eps
    x_t_pos = sab * (pos - mean_pos) + somab * pos_eps
    xtp4 = jnp.concatenate([x_t_pos, tfeat], axis=1)

    xe = xe_ref[...]
    x_t_x = sab * lx_ref[...] + somab * xe

    pre = _dot(x_t_x, w1a_ref[...]) + _dot(xtp4, w1b_ref[...]) + b1_ref[...]
    h = jnp.maximum(pre, 0.0)

    xp = _dot(h, w2x_ref[...]) + b2x_ref[...]
    pp = _dot(h, w2p_ref[...]) + b2p_ref[...]

    ex = jnp.sum((xe - xp) ** 2)
    ep = jnp.sum((pos_eps - pp) ** 2)

    @pl.when(i == 0)
    def _init():
        ox_ref[...] = ex.reshape(1, 1)
        op_ref[...] = ep.reshape(1, 1)

    @pl.when(i != 0)
    def _acc():
        ox_ref[...] = ox_ref[...] + ex.reshape(1, 1)
        op_ref[...] = op_ref[...] + ep.reshape(1, 1)


def _main_pass(parts, t, lx, xe, p8, w1a, w1b4, b1, w2x, b2x, w2p, b2p,
               block):
    n = lx.shape[0]
    nb = n // block
    p = parts.shape[0]
    full = lambda *s: pl.BlockSpec(s, lambda i: (0,) * len(s))
    return pl.pallas_call(
        _main_body,
        grid=(nb,),
        in_specs=[
            full(p, NUM_GRAPHS, 8),
            full(NUM_GRAPHS, 1),
            full(256, 2),
            full(D_FEAT, HIDDEN),
            full(4, HIDDEN),
            full(1, HIDDEN),
            full(HIDDEN, D_FEAT),
            full(1, D_FEAT),
            full(HIDDEN, 3),
            full(1, 3),
            pl.BlockSpec((block, D_FEAT), lambda i: (i, 0)),
            pl.BlockSpec((block, D_FEAT), lambda i: (i, 0)),
            pl.BlockSpec((block, 8), lambda i: (i, 0)),
        ],
        out_specs=[
            pl.BlockSpec((1, 1), lambda i: (0, 0)),
            pl.BlockSpec((1, 1), lambda i: (0, 0)),
        ],
        out_shape=[
            jax.ShapeDtypeStruct((1, 1), jnp.float32),
            jax.ShapeDtypeStruct((1, 1), jnp.float32),
        ],
        scratch_shapes=[pltpu.VMEM((NUM_GRAPHS, 16), jnp.float32)],
        compiler_params=pltpu.CompilerParams(
            dimension_semantics=("arbitrary",)),
        interpret=_INTERPRET,
    )(parts, t, _SCHED, w1a, w1b4, b1, w2x, b2x, w2p, b2p, lx, xe, p8)


BLOCK = 2000


def kernel(ligand_x, ligand_pos, protein_x, protein_pos, x_eps, pos_eps_raw,
           W1, b1, W2x, b2x, W2pos, b2pos, ligand_batch, protein_batch, t):
    n = ligand_x.shape[0]
    segf = ligand_batch.astype(jnp.float32)[:, None]
    ones = jnp.ones((n, 1), jnp.float32)
    p8 = jnp.concatenate([ligand_pos, pos_eps_raw, ones, segf], axis=1)
    segs3 = segf.reshape(n // BLOCK, 1, BLOCK)

    parts = _segment_sums_tc(p8, segs3, BLOCK)

    w1a = W1[0:D_FEAT]
    w1b4 = W1[D_FEAT:D_FEAT + 4]
    ox, op = _main_pass(parts, t, ligand_x, x_eps, p8,
                        w1a, w1b4, b1[None, :], W2x, b2x[None, :],
                        W2pos, b2pos[None, :], BLOCK)

    sum_x = ox[0, 0]
    sum_pos = op[0, 0]
    L_x = sum_x / (n * D_FEAT)
    L_pos = sum_pos / (n * 3)
    L_simple = 0.25 * (L_pos + L_x)
    L_unweighted = 0.5 * (sum_x + sum_pos) / (n * (D_FEAT + 3))
    return (L_simple, L_unweighted, L_pos, L_x)


# speedup vs baseline: 15.7393x; 3.9457x over previous
import functools

import jax
import jax.numpy as jnp
import numpy as np
from jax.experimental import pallas as pl
from jax.experimental.pallas import tpu as pltpu

T = 200
NUM_GRAPHS = 512
D_FEAT = 128
HIDDEN = 64

_INTERPRET = False

def _sched_table():
    tt = np.arange(T + 1, dtype=np.float64)
    alpha_bar = (1.0 - (tt / T) ** 2.0) ** 2
    alpha_bar = np.clip(alpha_bar, 1e-4, 1.0)
    alpha = np.clip(alpha_bar[1:] / alpha_bar[:-1], 1e-3, 1.0)
    alpha_bar = np.cumprod(alpha)
    sab = np.sqrt(alpha_bar)
    somab = np.sqrt(1.0 - alpha_bar)
    out = np.zeros((256, 2), np.float32)
    out[:T, 0] = sab
    out[:T, 1] = somab
    return jnp.asarray(out)

_SCHED = _sched_table()

_HI = jax.lax.Precision.DEFAULT


def _dot(a, b):
    return jnp.dot(a, b, precision=_HI, preferred_element_type=jnp.float32)


def _segsum_body(segs_ref, p8_ref, out_ref):
    i = pl.program_id(0)
    seg_row = segs_ref[...].reshape(1, -1)
    iota_g = jax.lax.broadcasted_iota(jnp.int32, (NUM_GRAPHS, 1), 0).astype(jnp.float32)
    onehot_t = (iota_g == seg_row).astype(jnp.float32)
    part = _dot(onehot_t, p8_ref[...])

    @pl.when(i == 0)
    def _init():
        out_ref[...] = part[None]

    @pl.when(i != 0)
    def _acc():
        out_ref[...] = out_ref[...] + part[None]


def _segment_sums_tc(p8, segs3, block):
    n = p8.shape[0]
    nb = n // block
    return pl.pallas_call(
        _segsum_body,
        grid=(nb,),
        in_specs=[
            pl.BlockSpec((1, 1, block), lambda i: (i, 0, 0)),
            pl.BlockSpec((block, 8), lambda i: (i, 0)),
        ],
        out_specs=pl.BlockSpec((1, NUM_GRAPHS, 8), lambda i: (0, 0, 0)),
        out_shape=jax.ShapeDtypeStruct((1, NUM_GRAPHS, 8), jnp.float32),
        compiler_params=pltpu.CompilerParams(
            dimension_semantics=("arbitrary",)),
        interpret=_INTERPRET,
    )(segs3, p8)


def _main_body(parts_ref, t_ref, sched_ref, w1a_ref, w1b_ref, b1_ref,
               w2x_ref, b2x_ref, w2p_ref, b2p_ref,
               lx_ref, xe_ref, p8_ref,
               ox_ref, op_ref, tbl_ref):
    i = pl.program_id(0)

    @pl.when(i == 0)
    def _finalize_table():
        sums = jnp.sum(parts_ref[...], axis=0)
        cnt = jnp.maximum(sums[:, 6:7], 1.0)
        means = sums[:, 0:6] / cnt
        t_f = t_ref[...].astype(jnp.float32)
        iota_t = jax.lax.broadcasted_iota(jnp.int32, (1, 256), 1).astype(jnp.float32)
        onehot_t = (t_f == iota_t).astype(jnp.float32)
        sch = _dot(onehot_t, sched_ref[...])
        tfeat = t_f * (1.0 / T)
        pad = jnp.zeros((NUM_GRAPHS, 7), jnp.float32)
        tbl_ref[...] = jnp.concatenate([means, sch, tfeat, pad], axis=1)

    p8 = p8_ref[...]
    seg_col = p8[:, 7:8]
    iota_g = jax.lax.broadcasted_iota(jnp.int32, (1, NUM_GRAPHS), 1).astype(jnp.float32)
    onehot = (seg_col == iota_g).astype(jnp.float32)
    vals = _dot(onehot, tbl_ref[...])

    mean_pos = vals[:, 0:3]
    mean_eps = vals[:, 3:6]
    sab = vals[:, 6:7]
    somab = vals[:, 7:8]
    tfeat = vals[:, 8:9]

    pos = p8[:, 0:3]
    eps = p8[:, 3:6]
    pos_eps = eps - mean_eps
    x_t_pos = sab * (pos - mean_pos) + somab * pos_eps
    xtp4 = jnp.concatenate([x_t_pos, tfeat], axis=1)

    xe = xe_ref[...]
    x_t_x = sab * lx_ref[...] + somab * xe

    pre = _dot(x_t_x, w1a_ref[...]) + _dot(xtp4, w1b_ref[...]) + b1_ref[...]
    h = jnp.maximum(pre, 0.0)

    xp = _dot(h, w2x_ref[...]) + b2x_ref[...]
    pp = _dot(h, w2p_ref[...]) + b2p_ref[...]

    ex = jnp.sum((xe - xp) ** 2)
    ep = jnp.sum((pos_eps - pp) ** 2)

    @pl.when(i == 0)
    def _init():
        ox_ref[...] = ex.reshape(1, 1)
        op_ref[...] = ep.reshape(1, 1)

    @pl.when(i != 0)
    def _acc():
        ox_ref[...] = ox_ref[...] + ex.reshape(1, 1)
        op_ref[...] = op_ref[...] + ep.reshape(1, 1)


def _main_pass(parts, t, lx, xe, p8, w1a, w1b4, b1, w2x, b2x, w2p, b2p,
               block):
    n = lx.shape[0]
    nb = n // block
    p = parts.shape[0]
    full = lambda *s: pl.BlockSpec(s, lambda i: (0,) * len(s))
    return pl.pallas_call(
        _main_body,
        grid=(nb,),
        in_specs=[
            full(p, NUM_GRAPHS, 8),
            full(NUM_GRAPHS, 1),
            full(256, 2),
            full(D_FEAT, HIDDEN),
            full(4, HIDDEN),
            full(1, HIDDEN),
            full(HIDDEN, D_FEAT),
            full(1, D_FEAT),
            full(HIDDEN, 3),
            full(1, 3),
            pl.BlockSpec((block, D_FEAT), lambda i: (i, 0)),
            pl.BlockSpec((block, D_FEAT), lambda i: (i, 0)),
            pl.BlockSpec((block, 8), lambda i: (i, 0)),
        ],
        out_specs=[
            pl.BlockSpec((1, 1), lambda i: (0, 0)),
            pl.BlockSpec((1, 1), lambda i: (0, 0)),
        ],
        out_shape=[
            jax.ShapeDtypeStruct((1, 1), jnp.float32),
            jax.ShapeDtypeStruct((1, 1), jnp.float32),
        ],
        scratch_shapes=[pltpu.VMEM((NUM_GRAPHS, 16), jnp.float32)],
        compiler_params=pltpu.CompilerParams(
            dimension_semantics=("arbitrary",)),
        interpret=_INTERPRET,
    )(parts, t, _SCHED, w1a, w1b4, b1, w2x, b2x, w2p, b2p, lx, xe, p8)


BLOCK = 5000


def kernel(ligand_x, ligand_pos, protein_x, protein_pos, x_eps, pos_eps_raw,
           W1, b1, W2x, b2x, W2pos, b2pos, ligand_batch, protein_batch, t):
    n = ligand_x.shape[0]
    segf = ligand_batch.astype(jnp.float32)[:, None]
    ones = jnp.ones((n, 1), jnp.float32)
    p8 = jnp.concatenate([ligand_pos, pos_eps_raw, ones, segf], axis=1)
    segs3 = segf.reshape(n // BLOCK, 1, BLOCK)

    parts = _segment_sums_tc(p8, segs3, BLOCK)

    w1a = W1[0:D_FEAT]
    w1b4 = W1[D_FEAT:D_FEAT + 4]
    ox, op = _main_pass(parts, t, ligand_x, x_eps, p8,
                        w1a, w1b4, b1[None, :], W2x, b2x[None, :],
                        W2pos, b2pos[None, :], BLOCK)

    sum_x = ox[0, 0]
    sum_pos = op[0, 0]
    L_x = sum_x / (n * D_FEAT)
    L_pos = sum_pos / (n * 3)
    L_simple = 0.25 * (L_pos + L_x)
    L_unweighted = 0.5 * (sum_x + sum_pos) / (n * (D_FEAT + 3))
    return (L_simple, L_unweighted, L_pos, L_x)
